# Initial kernel scaffold; baseline (speedup 1.0000x reference)
import jax
import jax.numpy as jnp
from jax.experimental import pallas as pl


def kernel(input1, input2, relative_coords_weight):
    raise NotImplementedError("write your pallas kernel here")


# speedup vs baseline: 4.5751x; 4.5751x over previous
import dataclasses
import functools

import jax
import jax.numpy as jnp
from jax import lax
from jax.experimental import pallas as pl
from jax.experimental.pallas import tpu as pltpu
from jax.experimental.pallas import tpu_sc as plsc

NC, NS, L = 2, 16, 16
NW = NC * NS


def _quantize(v1, v2):
    d = v1 - v2
    x = (jnp.minimum(jnp.maximum(d, -4.0), 4.0) + 4.0) * 10.0
    r = (x + 0.5).astype(jnp.int32)
    tie = (r.astype(jnp.float32) - x) == 0.5
    odd = (r & 1) == 1
    return jnp.where(tie & odd, r - 1, r)


def _sc_edge_vector(in1b, in2t, table, *, B, N1, N2, C):
    rows = B * N1
    rows_per_w = rows // NW
    row_len = N2 * C
    mesh = plsc.VectorSubcoreMesh(
        core_axis_name="core", subcore_axis_name="subcore",
        num_cores=NC, num_subcores=NS)
    cp = pltpu.CompilerParams()
    fields = pltpu.CompilerParams.__dataclass_fields__
    if "needs_layout_passes" in fields:
        cp = dataclasses.replace(cp, needs_layout_passes=False)
    if "use_tc_tiling_on_sc" in fields:
        cp = dataclasses.replace(cp, use_tc_tiling_on_sc=False)

    @functools.partial(
        pl.kernel,
        out_type=jax.ShapeDtypeStruct((rows * row_len, L), jnp.float32),
        mesh=mesh,
        scratch_types=[
            pltpu.VMEM((C, rows_per_w, L), jnp.float32),
            pltpu.VMEM((B * C, N2), jnp.float32),
            pltpu.VMEM((row_len,), jnp.int32),
            pltpu.VMEM((row_len, L), jnp.float32),
            pltpu.SemaphoreType.DMA,
        ],
        compiler_params=cp,
    )
    def k(in1b_hbm, in2t_hbm, table_hbm, out_hbm, in1_v, in2_v, idx_v,
          rows_v, sem):
        wid = lax.axis_index("subcore") * NC + lax.axis_index("core")
        r0 = wid * rows_per_w
        b = r0 // N1
        i0 = r0 % N1
        iota3 = lax.iota(jnp.int32, L) * C

        for c in range(C):
            pltpu.sync_copy(
                in1b_hbm.at[pl.ds((b * C + c) * N1 + i0, rows_per_w)],
                in1_v.at[c])
        pltpu.sync_copy(in2t_hbm, in2_v)

        @pl.loop(0, rows_per_w)
        def _(ri):
            for c in range(C):
                v1 = in1_v[c, ri]
                @pl.loop(0, N2 // L)
                def _(t):
                    v2 = in2_v[b * C + c, pl.ds(t * L, L)]
                    pos = iota3 + (t * (L * C) + c)
                    plsc.store_scatter(idx_v, [pos], _quantize(v1, v2))
            pltpu.async_copy(table_hbm.at[idx_v], rows_v, sem).wait()
            pltpu.sync_copy(rows_v, out_hbm.at[pl.ds((r0 + ri) * row_len,
                                                     row_len)])

    return k(in1b, in2t, table)


def kernel(input1, input2, relative_coords_weight):
    B, N1, C = input1.shape
    N2 = input2.shape[1]
    in1b = jnp.broadcast_to(
        jnp.transpose(input1, (0, 2, 1))[..., None],
        (B, C, N1, L)).reshape(B * C * N1, L)
    in2t = jnp.transpose(input2, (0, 2, 1)).reshape(B * C, N2)
    out = _sc_edge_vector(in1b, in2t, relative_coords_weight,
                          B=B, N1=N1, N2=N2, C=C)
    return out.reshape(B, N1, N2, C * L)


# speedup vs baseline: 11.1217x; 2.4309x over previous
import dataclasses
import functools

import jax
import jax.numpy as jnp
from jax import lax
from jax.experimental import pallas as pl
from jax.experimental.pallas import tpu as pltpu
from jax.experimental.pallas import tpu_sc as plsc

NC, NS, L = 2, 16, 16
NW = NC * NS


def _quantize(v1, v2):
    d = v1 - v2
    x = (jnp.minimum(jnp.maximum(d, -4.0), 4.0) + 4.0) * 10.0
    r = (x + 0.5).astype(jnp.int32)
    tie = (r.astype(jnp.float32) - x) == 0.5
    odd = (r & 1) == 1
    return jnp.where(tie & odd, r - 1, r)


def _sc_edge_vector(in1b, in2t, table, *, B, N1, N2, C):
    rows = B * N1
    rows_per_w = rows // NW
    row_len = N2 * C
    blk = row_len * L
    V = table.shape[0]
    mesh = plsc.VectorSubcoreMesh(
        core_axis_name="core", subcore_axis_name="subcore",
        num_cores=NC, num_subcores=NS)
    cp = pltpu.CompilerParams()
    fields = pltpu.CompilerParams.__dataclass_fields__
    if "needs_layout_passes" in fields:
        cp = dataclasses.replace(cp, needs_layout_passes=False)
    if "use_tc_tiling_on_sc" in fields:
        cp = dataclasses.replace(cp, use_tc_tiling_on_sc=False)

    @functools.partial(
        pl.kernel,
        out_type=jax.ShapeDtypeStruct((rows * blk,), jnp.float32),
        mesh=mesh,
        scratch_types=[
            pltpu.VMEM((V * L,), jnp.float32),
            pltpu.VMEM((C, rows_per_w, L), jnp.float32),
            pltpu.VMEM((B * C, N2), jnp.float32),
            pltpu.VMEM((blk,), jnp.float32),
            pltpu.VMEM((blk,), jnp.float32),
            pltpu.SemaphoreType.DMA,
            pltpu.SemaphoreType.DMA,
        ],
        compiler_params=cp,
    )
    def k(in1b_hbm, in2t_hbm, table_hbm, out_hbm, table_v, in1_v, in2_v,
          ob0, ob1, sem0, sem1):
        wid = lax.axis_index("subcore") * NC + lax.axis_index("core")
        r0 = wid * rows_per_w
        b = r0 // N1
        i0 = r0 % N1
        iota = lax.iota(jnp.int32, L)
        pat48 = iota * (L * C)
        kc = [jnp.full((L,), kk, jnp.int32) for kk in range(L)]

        pltpu.sync_copy(table_hbm, table_v)
        for c in range(C):
            pltpu.sync_copy(
                in1b_hbm.at[pl.ds((b * C + c) * N1 + i0, rows_per_w)],
                in1_v.at[c])
        pltpu.sync_copy(in2t_hbm, in2_v)

        def fill(ob, ri):
            for c in range(C):
                v1 = in1_v[c, ri]
                @pl.loop(0, N2 // L)
                def _(t):
                    q = _quantize(v1, in2_v[b * C + c, pl.ds(t * L, L)])
                    src = q * L
                    dst = pat48 + (t * (L * C * L) + c * L)
                    for kk in range(L):
                        col = plsc.load_gather(table_v, [src + kc[kk]])
                        plsc.store_scatter(ob, [dst + kc[kk]], col)

        def drain(ob, sem):
            pltpu.make_async_copy(
                ob, out_hbm.at[pl.ds(r0 * blk, blk)], sem).wait()

        obr = [ob0, ob1]
        sems = [sem0, sem1]

        @pl.loop(0, rows_per_w // 2)
        def _(g):
            for h in range(2):
                ob, sem = obr[h], sems[h]

                @pl.when(g > 0)
                def _():
                    drain(ob, sem)

                fill(ob, 2 * g + h)
                pltpu.async_copy(
                    ob,
                    out_hbm.at[pl.ds((r0 + 2 * g + h) * blk, blk)],
                    sem)

        drain(obr[0], sem0)
        drain(obr[1], sem1)

    return k(in1b, in2t, table.reshape(V * L))


def kernel(input1, input2, relative_coords_weight):
    B, N1, C = input1.shape
    N2 = input2.shape[1]
    in1b = jnp.broadcast_to(
        jnp.transpose(input1, (0, 2, 1))[..., None],
        (B, C, N1, L)).reshape(B * C * N1, L)
    in2t = jnp.transpose(input2, (0, 2, 1)).reshape(B * C, N2)
    out = _sc_edge_vector(in1b, in2t, relative_coords_weight,
                          B=B, N1=N1, N2=N2, C=C)
    return out.reshape(B, N1, N2, C * L)


# speedup vs baseline: 15.3592x; 1.3810x over previous
import dataclasses
import functools

import jax
import jax.numpy as jnp
from jax import lax
from jax.experimental import pallas as pl
from jax.experimental.pallas import tpu as pltpu
from jax.experimental.pallas import tpu_sc as plsc

NC, NS, L = 2, 16, 16
NW = NC * NS


def _quantize(v1, v2):
    d = v1 - v2
    x = (jnp.minimum(jnp.maximum(d, -4.0), 4.0) + 4.0) * 10.0
    r = (x + 0.5).astype(jnp.int32)
    tie = (r.astype(jnp.float32) - x) == 0.5
    odd = (r & 1) == 1
    return jnp.where(tie & odd, r - 1, r)


def _sc_edge_vector(in1b, in2t, table, *, B, N1, N2, C):
    rows = B * N1
    rows_per_w = rows // NW
    row_len = N2 * C
    blk = row_len * L
    V = table.shape[0]
    mesh = plsc.VectorSubcoreMesh(
        core_axis_name="core", subcore_axis_name="subcore",
        num_cores=NC, num_subcores=NS)
    cp = pltpu.CompilerParams()
    fields = pltpu.CompilerParams.__dataclass_fields__
    if "needs_layout_passes" in fields:
        cp = dataclasses.replace(cp, needs_layout_passes=False)
    if "use_tc_tiling_on_sc" in fields:
        cp = dataclasses.replace(cp, use_tc_tiling_on_sc=False)

    @functools.partial(
        pl.kernel,
        out_type=jax.ShapeDtypeStruct((rows * blk,), jnp.float32),
        mesh=mesh,
        scratch_types=[
            pltpu.VMEM((V * L,), jnp.float32),
            pltpu.VMEM((C, rows_per_w, L), jnp.float32),
            pltpu.VMEM((B * C, N2), jnp.float32),
            pltpu.VMEM((blk,), jnp.float32),
            pltpu.VMEM((blk,), jnp.float32),
            pltpu.SemaphoreType.DMA,
            pltpu.SemaphoreType.DMA,
        ],
        compiler_params=cp,
    )
    def k(in1b_hbm, in2t_hbm, table_hbm, out_hbm, table_v, in1_v, in2_v,
          ob0, ob1, sem0, sem1):
        wid = lax.axis_index("subcore") * NC + lax.axis_index("core")
        r0 = wid * rows_per_w
        b = r0 // N1
        i0 = r0 % N1
        iota = lax.iota(jnp.int32, L)
        pat48 = iota * (L * C)
        diag = [(iota + s) & (L - 1) for s in range(L)]

        pltpu.sync_copy(table_hbm, table_v)
        for c in range(C):
            pltpu.sync_copy(
                in1b_hbm.at[pl.ds((b * C + c) * N1 + i0, rows_per_w)],
                in1_v.at[c])
        pltpu.sync_copy(in2t_hbm, in2_v)

        def fill(ob, ri):
            for c in range(C):
                v1 = in1_v[c, ri]
                @pl.loop(0, N2 // L)
                def _(t):
                    q = _quantize(v1, in2_v[b * C + c, pl.ds(t * L, L)])
                    src = q * L
                    dst = pat48 + (t * (L * C * L) + c * L)
                    for s in range(L):
                        col = plsc.load_gather(table_v, [src + diag[s]])
                        plsc.store_scatter(ob, [dst + diag[s]], col)

        def drain(ob, sem):
            pltpu.make_async_copy(
                ob, out_hbm.at[pl.ds(r0 * blk, blk)], sem).wait()

        obr = [ob0, ob1]
        sems = [sem0, sem1]

        @pl.loop(0, rows_per_w // 2)
        def _(g):
            for h in range(2):
                ob, sem = obr[h], sems[h]

                @pl.when(g > 0)
                def _():
                    drain(ob, sem)

                fill(ob, 2 * g + h)
                pltpu.async_copy(
                    ob,
                    out_hbm.at[pl.ds((r0 + 2 * g + h) * blk, blk)],
                    sem)

        drain(obr[0], sem0)
        drain(obr[1], sem1)

    return k(in1b, in2t, table.reshape(V * L))


def kernel(input1, input2, relative_coords_weight):
    B, N1, C = input1.shape
    N2 = input2.shape[1]
    in1b = jnp.broadcast_to(
        jnp.transpose(input1, (0, 2, 1))[..., None],
        (B, C, N1, L)).reshape(B * C * N1, L)
    in2t = jnp.transpose(input2, (0, 2, 1)).reshape(B * C, N2)
    out = _sc_edge_vector(in1b, in2t, relative_coords_weight,
                          B=B, N1=N1, N2=N2, C=C)
    return out.reshape(B, N1, N2, C * L)


# speedup vs baseline: 16.0969x; 1.0480x over previous
import dataclasses
import functools

import jax
import jax.numpy as jnp
from jax import lax
from jax.experimental import pallas as pl
from jax.experimental.pallas import tpu as pltpu
from jax.experimental.pallas import tpu_sc as plsc

NC, NS, L = 2, 16, 16
NW = NC * NS


def _quantize(v1, v2):
    d = v1 - v2
    x = (jnp.minimum(jnp.maximum(d, -4.0), 4.0) + 4.0) * 10.0
    r = (x + 0.5).astype(jnp.int32)
    tie = (r.astype(jnp.float32) - x) == 0.5
    odd = (r & 1) == 1
    return jnp.where(tie & odd, r - 1, r)


def _sc_edge_vector(in1b, in2t, table, *, B, N1, N2, C):
    rows = B * N1
    rows_per_w = rows // NW
    row_len = N2 * C
    blk = row_len * L
    V = table.shape[0]
    mesh = plsc.VectorSubcoreMesh(
        core_axis_name="core", subcore_axis_name="subcore",
        num_cores=NC, num_subcores=NS)
    cp = pltpu.CompilerParams()
    fields = pltpu.CompilerParams.__dataclass_fields__
    if "needs_layout_passes" in fields:
        cp = dataclasses.replace(cp, needs_layout_passes=False)
    if "use_tc_tiling_on_sc" in fields:
        cp = dataclasses.replace(cp, use_tc_tiling_on_sc=False)

    @functools.partial(
        pl.kernel,
        out_type=jax.ShapeDtypeStruct((rows * blk,), jnp.float32),
        mesh=mesh,
        scratch_types=[
            pltpu.VMEM((V * L,), jnp.float32),
            pltpu.VMEM((C, rows_per_w, L), jnp.float32),
            pltpu.VMEM((B * C, N2), jnp.float32),
            pltpu.VMEM((blk,), jnp.float32),
            pltpu.VMEM((blk,), jnp.float32),
            pltpu.SemaphoreType.DMA,
            pltpu.SemaphoreType.DMA,
        ],
        compiler_params=cp,
    )
    def k(in1b_hbm, in2t_hbm, table_hbm, out_hbm, table_v, in1_v, in2_v,
          ob0, ob1, sem0, sem1):
        wid = lax.axis_index("subcore") * NC + lax.axis_index("core")
        r0 = wid * rows_per_w
        b = r0 // N1
        i0 = r0 % N1
        iota = lax.iota(jnp.int32, L)
        pat48 = iota * (L * C)
        diag = [(iota + s) & (L - 1) for s in range(L)]

        pltpu.sync_copy(table_hbm, table_v)
        for c in range(C):
            pltpu.sync_copy(
                in1b_hbm.at[pl.ds((b * C + c) * N1 + i0, rows_per_w)],
                in1_v.at[c])
        pltpu.sync_copy(in2t_hbm, in2_v)

        def fill(ob, ri):
            for c in range(C):
                v1 = in1_v[c, ri]
                @pl.loop(0, N2 // L // 2)
                def _(g2):
                    sd = []
                    for dt in range(2):
                        t = g2 * 2 + dt
                        q = _quantize(v1, in2_v[b * C + c, pl.ds(t * L, L)])
                        sd.append((q * L,
                                   pat48 + (t * (L * C * L) + c * L)))
                    for s in range(L):
                        for src, dst in sd:
                            col = plsc.load_gather(table_v, [src + diag[s]])
                            plsc.store_scatter(ob, [dst + diag[s]], col)

        def drain(ob, sem):
            pltpu.make_async_copy(
                ob, out_hbm.at[pl.ds(r0 * blk, blk)], sem).wait()

        obr = [ob0, ob1]
        sems = [sem0, sem1]

        @pl.loop(0, rows_per_w // 2)
        def _(g):
            for h in range(2):
                ob, sem = obr[h], sems[h]

                @pl.when(g > 0)
                def _():
                    drain(ob, sem)

                fill(ob, 2 * g + h)
                pltpu.async_copy(
                    ob,
                    out_hbm.at[pl.ds((r0 + 2 * g + h) * blk, blk)],
                    sem)

        drain(obr[0], sem0)
        drain(obr[1], sem1)

    return k(in1b, in2t, table.reshape(V * L))


def kernel(input1, input2, relative_coords_weight):
    B, N1, C = input1.shape
    N2 = input2.shape[1]
    in1b = jnp.broadcast_to(
        jnp.transpose(input1, (0, 2, 1))[..., None],
        (B, C, N1, L)).reshape(B * C * N1, L)
    in2t = jnp.transpose(input2, (0, 2, 1)).reshape(B * C, N2)
    out = _sc_edge_vector(in1b, in2t, relative_coords_weight,
                          B=B, N1=N1, N2=N2, C=C)
    return out.reshape(B, N1, N2, C * L)


# speedup vs baseline: 21.1095x; 1.3114x over previous
import dataclasses
import functools

import jax
import jax.numpy as jnp
from jax import lax
from jax.experimental import pallas as pl
from jax.experimental.pallas import tpu as pltpu
from jax.experimental.pallas import tpu_sc as plsc

NC, NS, L = 2, 16, 16
NW = NC * NS


def _quantize(v1, v2):
    d = v1 - v2
    x = (jnp.minimum(jnp.maximum(d, -4.0), 4.0) + 4.0) * 10.0
    r = (x + 0.5).astype(jnp.int32)
    tie = (r.astype(jnp.float32) - x) == 0.5
    odd = (r & 1) == 1
    return jnp.where(tie & odd, r - 1, r)


def _sc_edge_vector(in1b, in2t, table, *, B, N1, N2, C):
    rows = B * N1
    rows_per_w = rows // NW
    row_len = N2 * C
    blk = row_len * L
    V = table.shape[0]
    mesh = plsc.VectorSubcoreMesh(
        core_axis_name="core", subcore_axis_name="subcore",
        num_cores=NC, num_subcores=NS)
    cp = pltpu.CompilerParams()
    fields = pltpu.CompilerParams.__dataclass_fields__
    if "needs_layout_passes" in fields:
        cp = dataclasses.replace(cp, needs_layout_passes=False)
    if "use_tc_tiling_on_sc" in fields:
        cp = dataclasses.replace(cp, use_tc_tiling_on_sc=False)

    @functools.partial(
        pl.kernel,
        out_type=jax.ShapeDtypeStruct((rows * blk,), jnp.float32),
        mesh=mesh,
        scratch_types=[
            pltpu.VMEM((V * L,), jnp.float32),
            pltpu.VMEM((C, rows_per_w, L), jnp.float32),
            pltpu.VMEM((B * C, N2), jnp.float32),
            pltpu.VMEM((blk,), jnp.float32),
            pltpu.VMEM((blk,), jnp.float32),
            pltpu.SemaphoreType.DMA,
            pltpu.SemaphoreType.DMA,
        ],
        compiler_params=cp,
    )
    def k(in1b_hbm, in2t_hbm, table_hbm, out_hbm, table_v, in1_v, in2_v,
          ob0, ob1, sem0, sem1):
        wid = lax.axis_index("subcore") * NC + lax.axis_index("core")
        r0 = wid * rows_per_w
        b = r0 // N1
        i0 = r0 % N1
        iota = lax.iota(jnp.int32, L)
        pat48 = iota * (L * C)
        diag = [(iota + s) & (L - 1) for s in range(L)]

        pltpu.sync_copy(table_hbm, table_v)
        for c in range(C):
            pltpu.sync_copy(
                in1b_hbm.at[pl.ds((b * C + c) * N1 + i0, rows_per_w)],
                in1_v.at[c])
        pltpu.sync_copy(in2t_hbm, in2_v)

        def fill(ob, ri):
            for c in range(C):
                v1 = in1_v[c, ri]
                @plsc.parallel_loop(0, N2 // L, unroll=2)
                def _(t):
                    q = _quantize(v1, in2_v[b * C + c, pl.ds(t * L, L)])
                    src = q * L
                    dst = pat48 + (t * (L * C * L) + c * L)
                    for s in range(L):
                        col = plsc.load_gather(table_v, [src + diag[s]])
                        plsc.store_scatter(ob, [dst + diag[s]], col)

        def drain(ob, sem):
            pltpu.make_async_copy(
                ob, out_hbm.at[pl.ds(r0 * blk, blk)], sem).wait()

        obr = [ob0, ob1]
        sems = [sem0, sem1]

        @pl.loop(0, rows_per_w // 2)
        def _(g):
            for h in range(2):
                ob, sem = obr[h], sems[h]

                @pl.when(g > 0)
                def _():
                    drain(ob, sem)

                fill(ob, 2 * g + h)
                pltpu.async_copy(
                    ob,
                    out_hbm.at[pl.ds((r0 + 2 * g + h) * blk, blk)],
                    sem)

        drain(obr[0], sem0)
        drain(obr[1], sem1)

    return k(in1b, in2t, table.reshape(V * L))


def kernel(input1, input2, relative_coords_weight):
    B, N1, C = input1.shape
    N2 = input2.shape[1]
    in1b = jnp.broadcast_to(
        jnp.transpose(input1, (0, 2, 1))[..., None],
        (B, C, N1, L)).reshape(B * C * N1, L)
    in2t = jnp.transpose(input2, (0, 2, 1)).reshape(B * C, N2)
    out = _sc_edge_vector(in1b, in2t, relative_coords_weight,
                          B=B, N1=N1, N2=N2, C=C)
    return out.reshape(B, N1, N2, C * L)


# speedup vs baseline: 22.3467x; 1.0586x over previous
import dataclasses
import functools

import jax
import jax.numpy as jnp
from jax import lax
from jax.experimental import pallas as pl
from jax.experimental.pallas import tpu as pltpu
from jax.experimental.pallas import tpu_sc as plsc

NC, NS, L = 2, 16, 16
NW = NC * NS


def _quantize(v1, v2):
    d = v1 - v2
    x = (jnp.minimum(jnp.maximum(d, -4.0), 4.0) + 4.0) * 10.0
    r = (x + 0.5).astype(jnp.int32)
    tie = (r.astype(jnp.float32) - x) == 0.5
    odd = (r & 1) == 1
    return jnp.where(tie & odd, r - 1, r)


def _sc_edge_vector(in1b, in2t, table, *, B, N1, N2, C):
    rows = B * N1
    rows_per_w = rows // NW
    row_len = N2 * C
    blk = row_len * L
    V = table.shape[0]
    mesh = plsc.VectorSubcoreMesh(
        core_axis_name="core", subcore_axis_name="subcore",
        num_cores=NC, num_subcores=NS)
    cp = pltpu.CompilerParams()
    fields = pltpu.CompilerParams.__dataclass_fields__
    if "needs_layout_passes" in fields:
        cp = dataclasses.replace(cp, needs_layout_passes=False)
    if "use_tc_tiling_on_sc" in fields:
        cp = dataclasses.replace(cp, use_tc_tiling_on_sc=False)

    @functools.partial(
        pl.kernel,
        out_type=jax.ShapeDtypeStruct((rows * blk,), jnp.float32),
        mesh=mesh,
        scratch_types=[
            pltpu.VMEM((V * L,), jnp.float32),
            pltpu.VMEM((C, rows_per_w, L), jnp.float32),
            pltpu.VMEM((B * C, N2), jnp.float32),
            pltpu.VMEM((blk,), jnp.float32),
            pltpu.VMEM((blk,), jnp.float32),
            pltpu.SemaphoreType.DMA,
            pltpu.SemaphoreType.DMA,
        ],
        compiler_params=cp,
    )
    def k(in1b_hbm, in2t_hbm, table_hbm, out_hbm, table_v, in1_v, in2_v,
          ob0, ob1, sem0, sem1):
        wid = lax.axis_index("subcore") * NC + lax.axis_index("core")
        r0 = wid * rows_per_w
        b = r0 // N1
        i0 = r0 % N1
        iota = lax.iota(jnp.int32, L)
        pat48 = iota * (L * C)
        diag = [(iota + s) & (L - 1) for s in range(L)]

        pltpu.sync_copy(table_hbm, table_v)
        for c in range(C):
            pltpu.sync_copy(
                in1b_hbm.at[pl.ds((b * C + c) * N1 + i0, rows_per_w)],
                in1_v.at[c])
        pltpu.sync_copy(in2t_hbm, in2_v)

        def fill(ob, ri):
            for c in range(C):
                v1 = in1_v[c, ri]
                @plsc.parallel_loop(0, N2 // L, unroll=4)
                def _(t):
                    q = _quantize(v1, in2_v[b * C + c, pl.ds(t * L, L)])
                    src = q * L
                    dst = pat48 + (t * (L * C * L) + c * L)
                    for s in range(L):
                        col = plsc.load_gather(table_v, [src + diag[s]])
                        plsc.store_scatter(ob, [dst + diag[s]], col)

        def drain(ob, sem):
            pltpu.make_async_copy(
                ob, out_hbm.at[pl.ds(r0 * blk, blk)], sem).wait()

        obr = [ob0, ob1]
        sems = [sem0, sem1]

        @pl.loop(0, rows_per_w // 2)
        def _(g):
            for h in range(2):
                ob, sem = obr[h], sems[h]

                @pl.when(g > 0)
                def _():
                    drain(ob, sem)

                fill(ob, 2 * g + h)
                pltpu.async_copy(
                    ob,
                    out_hbm.at[pl.ds((r0 + 2 * g + h) * blk, blk)],
                    sem)

        drain(obr[0], sem0)
        drain(obr[1], sem1)

    return k(in1b, in2t, table.reshape(V * L))


def kernel(input1, input2, relative_coords_weight):
    B, N1, C = input1.shape
    N2 = input2.shape[1]
    in1b = jnp.broadcast_to(
        jnp.transpose(input1, (0, 2, 1))[..., None],
        (B, C, N1, L)).reshape(B * C * N1, L)
    in2t = jnp.transpose(input2, (0, 2, 1)).reshape(B * C, N2)
    out = _sc_edge_vector(in1b, in2t, relative_coords_weight,
                          B=B, N1=N1, N2=N2, C=C)
    return out.reshape(B, N1, N2, C * L)
